# baseline (speedup 1.0000x reference)
import functools

import jax
import jax.numpy as jnp
from jax import lax
from jax.experimental import pallas as pl
from jax.experimental.pallas import tpu as pltpu
from jax.experimental.pallas import tpu_sc as plsc

VOCAB = 1000000
HIDDEN = 64
BATCH = 4096
SEQ = 50

NC = 2
NS = 16
NW = NC * NS
L = 16

B_PER_W = BATCH // NW


def _make_kernel():
    mesh = plsc.VectorSubcoreMesh(core_axis_name="c", subcore_axis_name="s")

    @functools.partial(
        pl.kernel,
        out_type=jax.ShapeDtypeStruct((SEQ, HIDDEN, BATCH), jnp.float32),
        mesh=mesh,
        scratch_types=[
            pltpu.VMEM((B_PER_W, SEQ), jnp.int32),
            pltpu.VMEM((SEQ, B_PER_W), jnp.int32),
            pltpu.VMEM((B_PER_W, HIDDEN), jnp.float32),
            pltpu.VMEM((B_PER_W, HIDDEN), jnp.float32),
            pltpu.VMEM((HIDDEN, B_PER_W), jnp.float32),
            pltpu.VMEM((HIDDEN, B_PER_W), jnp.float32),
            pltpu.SemaphoreType.DMA,
            pltpu.SemaphoreType.DMA,
            pltpu.SemaphoreType.DMA,
            pltpu.SemaphoreType.DMA,
        ],
        compiler_params=pltpu.CompilerParams(
            use_tc_tiling_on_sc=False, needs_layout_passes=False),
    )
    def embed(ids_hbm, table_hbm, out_hbm,
              ids_v, idsT_v, in0, in1, ot0, ot1, g0, g1, o0, o1):
        wid = lax.axis_index("s") * NC + lax.axis_index("c")
        base = wid * B_PER_W
        pltpu.sync_copy(ids_hbm.at[pl.ds(base, B_PER_W)], ids_v)

        lanes = lax.iota(jnp.int32, L)

        def tr_ids(s, carry):
            for g in range(B_PER_W // L):
                col = plsc.load_gather(
                    ids_v, [g * L + lanes, jnp.full((L,), s, jnp.int32)])
                idsT_v[s, pl.ds(g * L, L)] = col
            return carry

        lax.fori_loop(0, SEQ, tr_ids, 0)

        ibufs = (in0, in1)
        obufs = (ot0, ot1)
        gsems = (g0, g1)
        osems = (o0, o1)

        def gather_s(s, p):
            pltpu.async_copy(table_hbm.at[idsT_v.at[s]], ibufs[p], gsems[p])

        def wait_gather(p):
            pltpu.make_async_copy(
                table_hbm.at[pl.ds(0, B_PER_W)], ibufs[p], gsems[p]).wait()

        gather_s(0, 0)
        gather_s(1, 1)

        def half(t, p, first, last):
            s = 2 * t + p
            wait_gather(p)

            @pl.when(jnp.logical_not(first))
            def _():
                pltpu.make_async_copy(
                    obufs[p], out_hbm.at[0, :, pl.ds(0, B_PER_W)],
                    osems[p]).wait()

            def tr_c(c, carry):
                for g in range(B_PER_W // L):
                    v = plsc.load_gather(
                        ibufs[p],
                        [g * L + lanes, jnp.full((L,), c, jnp.int32)])
                    obufs[p][c, pl.ds(g * L, L)] = v
                return carry

            lax.fori_loop(0, HIDDEN, tr_c, 0)

            pltpu.async_copy(
                obufs[p], out_hbm.at[s, :, pl.ds(base, B_PER_W)], osems[p])

            @pl.when(jnp.logical_not(last))
            def _():
                gather_s(s + 2, p)

        def body(t, carry):
            first = t == 0
            last = t == (SEQ // 2 - 1)
            half(t, 0, first, last)
            half(t, 1, first, last)
            return carry

        lax.fori_loop(0, SEQ // 2, body, 0)

        pltpu.make_async_copy(
            ot0, out_hbm.at[0, :, pl.ds(0, B_PER_W)], o0).wait()
        pltpu.make_async_copy(
            ot1, out_hbm.at[0, :, pl.ds(0, B_PER_W)], o1).wait()

    return embed


_EMBED = _make_kernel()


def kernel(input_ids, word_embeddings):
    out_t = _EMBED(input_ids.astype(jnp.int32), word_embeddings)
    return jnp.transpose(out_t, (2, 0, 1))


# speedup vs baseline: 1.1115x; 1.1115x over previous
import functools

import jax
import jax.numpy as jnp
from jax import lax
from jax.experimental import pallas as pl
from jax.experimental.pallas import tpu as pltpu
from jax.experimental.pallas import tpu_sc as plsc

VOCAB = 1000000
HIDDEN = 64
BATCH = 4096
SEQ = 50

NC = 2
NS = 16
NW = NC * NS
L = 16

B_PER_W = BATCH // NW
Q = 32
NQ = B_PER_W // Q


def _make_kernel():
    mesh = plsc.VectorSubcoreMesh(core_axis_name="c", subcore_axis_name="s")

    @functools.partial(
        pl.kernel,
        out_type=jax.ShapeDtypeStruct((SEQ, HIDDEN, BATCH), jnp.float32),
        mesh=mesh,
        scratch_types=[
            pltpu.VMEM((B_PER_W, SEQ), jnp.int32),
            pltpu.VMEM((SEQ, B_PER_W), jnp.int32),
            pltpu.VMEM((Q, 8, HIDDEN), jnp.float32),
            pltpu.VMEM((Q, 8, HIDDEN), jnp.float32),
            pltpu.VMEM((HIDDEN, B_PER_W), jnp.float32),
            pltpu.SemaphoreType.DMA,
            pltpu.SemaphoreType.DMA,
            pltpu.SemaphoreType.DMA,
        ],
        compiler_params=pltpu.CompilerParams(needs_layout_passes=False),
    )
    def embed(ids_hbm, table_hbm, out_hbm,
              ids_v, idsT_v, blk0, blk1, oslab, g0, g1, osem):
        wid = lax.axis_index("s") * NC + lax.axis_index("c")
        base = wid * B_PER_W
        pltpu.sync_copy(ids_hbm.at[pl.ds(base, B_PER_W)], ids_v)

        lanes = lax.iota(jnp.int32, L)

        def tr_ids(s, carry):
            for g in range(B_PER_W // L):
                col = plsc.load_gather(
                    ids_v, [g * L + lanes, jnp.full((L,), s, jnp.int32)])
                idsT_v[s, pl.ds(g * L, L)] = col
            return carry

        lax.fori_loop(0, SEQ, tr_ids, 0)

        blks = (blk0, blk1)
        gsems = (g0, g1)

        def fetch_quarter(s, q, p):
            offs = []
            for g in range(Q // L):
                v = idsT_v[s, pl.ds(q * Q + g * L, L)]
                offs.append(v % 8)
                blk = (v // 8) * 8
                for j in range(L):
                    pltpu.async_copy(
                        table_hbm.at[pl.ds(pl.multiple_of(blk[j], 8), 8)],
                        blks[p].at[g * L + j],
                        gsems[p])
            return offs

        def drain_quarter(p):
            for _ in range(Q):
                pltpu.make_async_copy(
                    table_hbm.at[pl.ds(0, 8)], blks[p].at[0], gsems[p]).wait()

        def transpose_quarter(q, p, offs):
            def tr_c(c8, carry):
                o0, o1 = carry
                c0 = c8 * 8
                for cc in range(8):
                    c = c0 + cc
                    cvec = jnp.full((L,), c, jnp.int32)
                    for g, ov in enumerate((o0, o1)):
                        v = plsc.load_gather(
                            blks[p], [g * L + lanes, ov, cvec])
                        oslab[c, pl.ds(q * Q + g * L, L)] = v
                return o0, o1

            lax.fori_loop(0, HIDDEN // 8, tr_c, (offs[0], offs[1]))

        def step(s, carry):
            offs0 = fetch_quarter(s, 0, 0)
            offs1 = fetch_quarter(s, 1, 1)
            drain_quarter(0)
            transpose_quarter(0, 0, offs0)
            offs2 = fetch_quarter(s, 2, 0)
            drain_quarter(1)
            transpose_quarter(1, 1, offs1)
            offs3 = fetch_quarter(s, 3, 1)
            drain_quarter(0)
            transpose_quarter(2, 0, offs2)
            drain_quarter(1)
            transpose_quarter(3, 1, offs3)
            pltpu.async_copy(
                oslab, out_hbm.at[s, :, pl.ds(base, B_PER_W)], osem)
            pltpu.make_async_copy(
                oslab, out_hbm.at[0, :, pl.ds(0, B_PER_W)], osem).wait()
            return carry

        lax.fori_loop(0, SEQ, step, 0)

    return embed


_EMBED = _make_kernel()


def kernel(input_ids, word_embeddings):
    out_t = _EMBED(input_ids.astype(jnp.int32), word_embeddings)
    return jnp.transpose(out_t, (2, 0, 1))
